# baseline (speedup 1.0000x reference)
import functools

import jax
import jax.numpy as jnp
from jax import lax
from jax.experimental import pallas as pl
from jax.experimental.pallas import tpu as pltpu
from jax.experimental.pallas import tpu_sc as plsc

NUM_SC_CORES = 2
NUM_SC_SUBCORES = 16
NUM_WORKERS = NUM_SC_CORES * NUM_SC_SUBCORES

M_BLK = 2000


def _tc_body(x_ref, w_ref, q_ref, mm_ref, oq_ref, ow_ref):
    mm_ref[...] = jnp.dot(x_ref[...], w_ref[...],
                          preferred_element_type=jnp.float32)
    oq_ref[...] = q_ref[...]
    ow_ref[...] = w_ref[...]


def _tc_matmul_and_item_copies(item_content, W, Q):
    M, K = item_content.shape
    N = W.shape[1]
    grid = (M // M_BLK,)
    return pl.pallas_call(
        _tc_body,
        grid=grid,
        in_specs=[
            pl.BlockSpec((M_BLK, K), lambda i: (i, 0)),
            pl.BlockSpec((K, N), lambda i: (0, 0)),
            pl.BlockSpec((M_BLK, N), lambda i: (i, 0)),
        ],
        out_specs=[
            pl.BlockSpec((M_BLK, N), lambda i: (i, 0)),
            pl.BlockSpec((M_BLK, N), lambda i: (i, 0)),
            pl.BlockSpec((K, N), lambda i: (0, 0)),
        ],
        out_shape=[
            jax.ShapeDtypeStruct((M, N), jnp.float32),
            jax.ShapeDtypeStruct((M, N), jnp.float32),
            jax.ShapeDtypeStruct((K, N), jnp.float32),
        ],
    )(item_content, W, Q)


def _sc_copy_body(p_hbm, pq2_hbm, op_hbm, opq2_hbm, sem1, sem2):
    wid = lax.axis_index("s") * NUM_SC_CORES + lax.axis_index("c")
    n = p_hbm.shape[0]
    rows = (n // NUM_WORKERS) // 8 * 8
    tail = n - rows * NUM_WORKERS
    base = wid * rows
    c1 = pltpu.async_copy(p_hbm.at[pl.ds(base, rows)],
                          op_hbm.at[pl.ds(base, rows)], sem1)
    c2 = pltpu.async_copy(pq2_hbm.at[pl.ds(base, rows)],
                          opq2_hbm.at[pl.ds(base, rows)], sem2)
    if tail:
        @pl.when(wid == NUM_WORKERS - 1)
        def _copy_tail():
            t = rows * NUM_WORKERS
            pltpu.async_copy(p_hbm.at[pl.ds(t, tail)],
                             op_hbm.at[pl.ds(t, tail)], sem1).wait()
            pltpu.async_copy(pq2_hbm.at[pl.ds(t, tail)],
                             opq2_hbm.at[pl.ds(t, tail)], sem2).wait()
    c1.wait()
    c2.wait()


def _sc_copy_tables(P, PQ2):
    mesh = plsc.VectorSubcoreMesh(core_axis_name="c", subcore_axis_name="s")
    fn = pl.kernel(
        _sc_copy_body,
        out_type=[
            jax.ShapeDtypeStruct(P.shape, P.dtype),
            jax.ShapeDtypeStruct(PQ2.shape, PQ2.dtype),
        ],
        mesh=mesh,
        scratch_types=[pltpu.SemaphoreType.DMA, pltpu.SemaphoreType.DMA],
    )
    return fn(P, PQ2)


def kernel(P, Q, PQ2, item_content, W):
    oP, oPQ2 = _sc_copy_tables(P, PQ2)
    item_emb2, oQ, oW = _tc_matmul_and_item_copies(item_content, W, Q)
    return (oP, oQ, oPQ2, item_emb2, oW)


# speedup vs baseline: 14.4185x; 14.4185x over previous
import functools

import jax
import jax.numpy as jnp
from jax import lax
from jax.experimental import pallas as pl
from jax.experimental.pallas import tpu as pltpu
from jax.experimental.pallas import tpu_sc as plsc

NUM_SC_CORES = 2
NUM_SC_SUBCORES = 16
NUM_WORKERS = NUM_SC_CORES * NUM_SC_SUBCORES

M_BLK = 2000


def _tc_body(x_ref, w_ref, q_ref, mm_ref, oq_ref, ow_ref):
    mm_ref[...] = jnp.dot(x_ref[...], w_ref[...],
                          preferred_element_type=jnp.float32)
    oq_ref[...] = q_ref[...]
    ow_ref[...] = w_ref[...]


def _tc_matmul_and_item_copies(item_content, W, Q):
    M, K = item_content.shape
    N = W.shape[1]
    grid = (M // M_BLK,)
    return pl.pallas_call(
        _tc_body,
        grid=grid,
        in_specs=[
            pl.BlockSpec((M_BLK, K), lambda i: (i, 0)),
            pl.BlockSpec((K, N), lambda i: (0, 0)),
            pl.BlockSpec((M_BLK, N), lambda i: (i, 0)),
        ],
        out_specs=[
            pl.BlockSpec((M_BLK, N), lambda i: (i, 0)),
            pl.BlockSpec((M_BLK, N), lambda i: (i, 0)),
            pl.BlockSpec((K, N), lambda i: (0, 0)),
        ],
        out_shape=[
            jax.ShapeDtypeStruct((M, N), jnp.float32),
            jax.ShapeDtypeStruct((M, N), jnp.float32),
            jax.ShapeDtypeStruct((K, N), jnp.float32),
        ],
    )(item_content, W, Q)


CH = 504


def _staged_table_copy(src, dst, base, nch, bufs, sin, sout):
    def wait_in(b):
        pltpu.make_async_copy(src.at[pl.ds(0, CH)], bufs[b], sin[b]).wait()

    def wait_out(b):
        pltpu.make_async_copy(bufs[b], dst.at[pl.ds(0, CH)], sout[b]).wait()

    def step(i, b):
        @pl.when(i >= 2)
        def _():
            wait_out(b)
        pltpu.async_copy(src.at[pl.ds(base + i * CH, CH)], bufs[b], sin[b])
        wait_in(b)
        pltpu.async_copy(bufs[b], dst.at[pl.ds(base + i * CH, CH)], sout[b])

    def pair(j, carry):
        step(2 * j, 0)
        step(2 * j + 1, 1)
        return carry

    lax.fori_loop(0, nch // 2, pair, 0)
    wait_out(0)
    wait_out(1)


def _sc_copy_body(p_hbm, pq2_hbm, op_hbm, opq2_hbm, b0, b1,
                  si0, si1, so0, so1):
    wid = lax.axis_index("s") * NUM_SC_CORES + lax.axis_index("c")
    n = p_hbm.shape[0]
    rows = (n // NUM_WORKERS) // CH * CH
    tail = n - rows * NUM_WORKERS
    base = wid * rows
    bufs = (b0, b1)
    sin = (si0, si1)
    sout = (so0, so1)
    _staged_table_copy(p_hbm, op_hbm, base, rows // CH, bufs, sin, sout)
    _staged_table_copy(pq2_hbm, opq2_hbm, base, rows // CH, bufs, sin, sout)
    if tail:
        @pl.when(wid == NUM_WORKERS - 1)
        def _copy_tail():
            t = rows * NUM_WORKERS
            for src, dst in ((p_hbm, op_hbm), (pq2_hbm, opq2_hbm)):
                pltpu.async_copy(src.at[pl.ds(t, tail)],
                                 b0.at[pl.ds(0, tail)], si0).wait()
                pltpu.async_copy(b0.at[pl.ds(0, tail)],
                                 dst.at[pl.ds(t, tail)], so0).wait()


def _sc_copy_tables(P, PQ2):
    mesh = plsc.VectorSubcoreMesh(core_axis_name="c", subcore_axis_name="s")
    fn = pl.kernel(
        _sc_copy_body,
        out_type=[
            jax.ShapeDtypeStruct(P.shape, P.dtype),
            jax.ShapeDtypeStruct(PQ2.shape, PQ2.dtype),
        ],
        mesh=mesh,
        scratch_types=[
            pltpu.VMEM((CH, 64), jnp.float32),
            pltpu.VMEM((CH, 64), jnp.float32),
            pltpu.SemaphoreType.DMA,
            pltpu.SemaphoreType.DMA,
            pltpu.SemaphoreType.DMA,
            pltpu.SemaphoreType.DMA,
        ],
    )
    return fn(P, PQ2)


def kernel(P, Q, PQ2, item_content, W):
    oP, oPQ2 = _sc_copy_tables(P, PQ2)
    item_emb2, oQ, oW = _tc_matmul_and_item_copies(item_content, W, Q)
    return (oP, oQ, oPQ2, item_emb2, oW)


# speedup vs baseline: 14.4204x; 1.0001x over previous
import functools

import jax
import jax.numpy as jnp
from jax import lax
from jax.experimental import pallas as pl
from jax.experimental.pallas import tpu as pltpu
from jax.experimental.pallas import tpu_sc as plsc

NUM_SC_CORES = 2
NUM_SC_SUBCORES = 16
NUM_WORKERS = NUM_SC_CORES * NUM_SC_SUBCORES

M_BLK = 2000


def _tc_body(x_ref, w_ref, q_ref, mm_ref, oq_ref, ow_ref):
    mm_ref[...] = jnp.dot(x_ref[...], w_ref[...],
                          preferred_element_type=jnp.float32)
    oq_ref[...] = q_ref[...]
    ow_ref[...] = w_ref[...]


def _tc_matmul_and_item_copies(item_content, W, Q):
    M, K = item_content.shape
    N = W.shape[1]
    grid = (M // M_BLK,)
    return pl.pallas_call(
        _tc_body,
        grid=grid,
        in_specs=[
            pl.BlockSpec((M_BLK, K), lambda i: (i, 0)),
            pl.BlockSpec((K, N), lambda i: (0, 0)),
            pl.BlockSpec((M_BLK, N), lambda i: (i, 0)),
        ],
        out_specs=[
            pl.BlockSpec((M_BLK, N), lambda i: (i, 0)),
            pl.BlockSpec((M_BLK, N), lambda i: (i, 0)),
            pl.BlockSpec((K, N), lambda i: (0, 0)),
        ],
        out_shape=[
            jax.ShapeDtypeStruct((M, N), jnp.float32),
            jax.ShapeDtypeStruct((M, N), jnp.float32),
            jax.ShapeDtypeStruct((K, N), jnp.float32),
        ],
    )(item_content, W, Q)


CH = 504


def _staged_table_copy(src, dst, base, nch, bufs, sin, sout):
    def wait_in(b):
        pltpu.make_async_copy(src.at[pl.ds(0, CH)], bufs[b], sin[b]).wait()

    def wait_out(b):
        pltpu.make_async_copy(bufs[b], dst.at[pl.ds(0, CH)], sout[b]).wait()

    def step(i, b):
        @pl.when(i >= 2)
        def _():
            wait_out(b)
        pltpu.async_copy(src.at[pl.ds(base + i * CH, CH)], bufs[b], sin[b])
        wait_in(b)
        pltpu.async_copy(bufs[b], dst.at[pl.ds(base + i * CH, CH)], sout[b])

    def pair(j, carry):
        step(2 * j, 0)
        step(2 * j + 1, 1)
        return carry

    lax.fori_loop(0, nch // 2, pair, 0)
    wait_out(0)
    wait_out(1)


def _sc_copy_body(p_hbm, pq2_hbm, op_hbm, opq2_hbm, b0, b1,
                  si0, si1, so0, so1):
    wid = lax.axis_index("s") * NUM_SC_CORES + lax.axis_index("c")
    n = p_hbm.shape[0]
    rows = (n // NUM_WORKERS) // CH * CH
    tail = n - rows * NUM_WORKERS
    base = wid * rows
    bufs = (b0, b1)
    sin = (si0, si1)
    sout = (so0, so1)
    _staged_table_copy(p_hbm, op_hbm, base, rows // CH, bufs, sin, sout)
    _staged_table_copy(pq2_hbm, opq2_hbm, base, rows // CH, bufs, sin, sout)
    if tail:
        @pl.when(wid == NUM_WORKERS - 1)
        def _copy_tail():
            t = rows * NUM_WORKERS
            for src, dst in ((p_hbm, op_hbm), (pq2_hbm, opq2_hbm)):
                pltpu.async_copy(src.at[pl.ds(t, tail)],
                                 b0.at[pl.ds(0, tail)], si0).wait()
                pltpu.async_copy(b0.at[pl.ds(0, tail)],
                                 dst.at[pl.ds(t, tail)], so0).wait()


def _sc_copy_tables(P, PQ2):
    mesh = plsc.VectorSubcoreMesh(core_axis_name="c", subcore_axis_name="s")
    fn = pl.kernel(
        _sc_copy_body,
        out_type=[
            jax.ShapeDtypeStruct(P.shape, P.dtype),
            jax.ShapeDtypeStruct(PQ2.shape, PQ2.dtype),
        ],
        mesh=mesh,
        compiler_params=pltpu.CompilerParams(use_tc_tiling_on_sc=True),
        scratch_types=[
            pltpu.VMEM((CH, 64), jnp.float32),
            pltpu.VMEM((CH, 64), jnp.float32),
            pltpu.SemaphoreType.DMA,
            pltpu.SemaphoreType.DMA,
            pltpu.SemaphoreType.DMA,
            pltpu.SemaphoreType.DMA,
        ],
    )
    return fn(P, PQ2)


def kernel(P, Q, PQ2, item_content, W):
    oP, oPQ2 = _sc_copy_tables(P, PQ2)
    item_emb2, oQ, oW = _tc_matmul_and_item_copies(item_content, W, Q)
    return (oP, oQ, oPQ2, item_emb2, oW)
